# baseline (speedup 1.0000x reference)
import functools

import jax
import jax.numpy as jnp
from jax.experimental import pallas as pl
from jax.experimental.pallas import tpu as pltpu

TB = 16


def _stitch_kernel(eid_ref, x_ref, Ws1_ref, bs1_ref, Ws2_ref, bs2_ref, out_ref):
    i = pl.program_id(0)
    for t in range(TB):
        e = eid_ref[i * TB + t]
        xt = x_ref[t]
        w1 = Ws1_ref[e]
        b1 = bs1_ref[pl.ds(e, 1)]
        h = jnp.dot(xt, w1, preferred_element_type=jnp.float32) + b1
        a = h / (1.0 + jnp.abs(h))
        w2 = Ws2_ref[e]
        b2 = bs2_ref[pl.ds(e, 1)]
        out_ref[t] = jnp.dot(a, w2, preferred_element_type=jnp.float32) + b2


@jax.jit
def kernel(x, Ws1, bs1, Ws2, bs2, eid):
    B, MAX_F, N = x.shape
    E, _, H = Ws1.shape
    P = Ws2.shape[-1]
    grid = B // TB

    grid_spec = pltpu.PrefetchScalarGridSpec(
        num_scalar_prefetch=1,
        grid=(grid,),
        in_specs=[
            pl.BlockSpec((TB, MAX_F, N), lambda i, eid_ref: (i, 0, 0)),
            pl.BlockSpec((E, N, H), lambda i, eid_ref: (0, 0, 0)),
            pl.BlockSpec((E, H), lambda i, eid_ref: (0, 0)),
            pl.BlockSpec((E, H, P), lambda i, eid_ref: (0, 0, 0)),
            pl.BlockSpec((E, P), lambda i, eid_ref: (0, 0)),
        ],
        out_specs=pl.BlockSpec((TB, MAX_F, P), lambda i, eid_ref: (i, 0, 0)),
    )
    return pl.pallas_call(
        _stitch_kernel,
        grid_spec=grid_spec,
        out_shape=jax.ShapeDtypeStruct((B, MAX_F, P), jnp.float32),
    )(eid, x, Ws1, bs1, Ws2, bs2)
